# Initial kernel scaffold; baseline (speedup 1.0000x reference)
import jax
import jax.numpy as jnp
from jax.experimental import pallas as pl


def kernel(grid, ptcloud):
    raise NotImplementedError("write your pallas kernel here")


# speedup vs baseline: 1.0427x; 1.0427x over previous
import functools

import jax
import jax.numpy as jnp
from jax import lax
from jax.experimental import pallas as pl
from jax.experimental.pallas import tpu as pltpu
from jax.experimental.pallas import tpu_sc as plsc

SCALE = 32
NG = 2 * SCALE
NG3 = NG * NG * NG

_TAP_OFFS = tuple(dx * NG * NG + dy * NG + dz
                  for dx in (0, 1) for dy in (0, 1) for dz in (0, 1))

L = 16
CHUNK = 2048


def _make_sampler(B, N):
    NW = 32
    assert B == NW and N % CHUNK == 0
    mesh = plsc.VectorSubcoreMesh(core_axis_name="c", subcore_axis_name="s")

    @functools.partial(
        pl.kernel,
        mesh=mesh,
        out_type=jax.ShapeDtypeStruct((B * N,), jnp.float32),
        scratch_types=[
            pltpu.VMEM((CHUNK,), jnp.float32),
            pltpu.VMEM((CHUNK,), jnp.float32),
            pltpu.VMEM((CHUNK,), jnp.float32),
            pltpu.VMEM((8 * CHUNK,), jnp.int32),
            pltpu.VMEM((8 * CHUNK,), jnp.float32),
            pltpu.VMEM((8 * CHUNK,), jnp.float32),
            pltpu.VMEM((CHUNK,), jnp.float32),
            pltpu.SemaphoreType.DMA,
        ],
    )
    def sampler(grid_hbm, x_hbm, y_hbm, z_hbm, out_hbm,
                xv, yv, zv, idxv, wv, valv, outv, sem):
        wid = lax.axis_index("s") * 2 + lax.axis_index("c")
        pbase = wid * N
        gbase = wid * NG3

        def chunk_body(ci, _):
            base = pbase + ci * CHUNK
            pltpu.sync_copy(x_hbm.at[pl.ds(base, CHUNK)], xv)
            pltpu.sync_copy(y_hbm.at[pl.ds(base, CHUNK)], yv)
            pltpu.sync_copy(z_hbm.at[pl.ds(base, CHUNK)], zv)

            def grp(i, _):
                o = i * L
                xs = xv[pl.ds(o, L)] + float(SCALE)
                ys = yv[pl.ds(o, L)] + float(SCALE)
                zs = zv[pl.ds(o, L)] + float(SCALE)
                fi = xs.astype(jnp.int32)
                fj = ys.astype(jnp.int32)
                fk = zs.astype(jnp.int32)
                tx = xs - fi.astype(jnp.float32)
                ty = ys - fj.astype(jnp.float32)
                tz = zs - fk.astype(jnp.float32)
                ii = jnp.minimum(fi, NG - 2)
                jj = jnp.minimum(fj, NG - 2)
                kk = jnp.minimum(fk, NG - 2)
                lin0 = (ii << 12) + (jj << 6) + kk + gbase
                sx = 1.0 - tx
                sy = 1.0 - ty
                sz = 1.0 - tz
                a00 = sx * sy
                a01 = sx * ty
                a10 = tx * sy
                a11 = tx * ty
                ws = (a00 * sz, a00 * tz, a01 * sz, a01 * tz,
                      a10 * sz, a10 * tz, a11 * sz, a11 * tz)
                for t in range(8):
                    idxv[pl.ds(t * CHUNK + o, L)] = lin0 + _TAP_OFFS[t]
                    wv[pl.ds(t * CHUNK + o, L)] = ws[t]

            lax.fori_loop(0, CHUNK // L, grp, None)

            pltpu.async_copy(grid_hbm.at[idxv], valv, sem).wait()

            def grp2(i, _):
                o = i * L
                acc = wv[pl.ds(o, L)] * valv[pl.ds(o, L)]
                for t in range(1, 8):
                    acc = acc + (wv[pl.ds(t * CHUNK + o, L)]
                                 * valv[pl.ds(t * CHUNK + o, L)])
                outv[pl.ds(o, L)] = acc

            lax.fori_loop(0, CHUNK // L, grp2, None)

            pltpu.sync_copy(outv, out_hbm.at[pl.ds(base, CHUNK)])

        lax.fori_loop(0, N // CHUNK, chunk_body, None)

    return sampler


def kernel(grid, ptcloud):
    B, N = ptcloud.shape[0], ptcloud.shape[1]
    x = ptcloud[..., 0].reshape(-1)
    y = ptcloud[..., 1].reshape(-1)
    z = ptcloud[..., 2].reshape(-1)
    gflat = grid.reshape(-1)
    out = _make_sampler(B, N)(gflat, x, y, z)
    return out.reshape(B, N)


# speedup vs baseline: 1.2350x; 1.1844x over previous
import functools

import jax
import jax.numpy as jnp
from jax import lax
from jax.experimental import pallas as pl
from jax.experimental.pallas import tpu as pltpu
from jax.experimental.pallas import tpu_sc as plsc

SCALE = 32
NG = 2 * SCALE
NG3 = NG * NG * NG

_TAP_OFFS = tuple(dx * NG * NG + dy * NG + dz
                  for dx in (0, 1) for dy in (0, 1) for dz in (0, 1))

L = 16
CHUNK = 2048
NBUF = 2


def _make_sampler(B, N):
    NW = 32
    assert B == NW and N % (CHUNK * NBUF) == 0
    nch = N // CHUNK
    mesh = plsc.VectorSubcoreMesh(core_axis_name="c", subcore_axis_name="s")

    @functools.partial(
        pl.kernel,
        mesh=mesh,
        out_type=jax.ShapeDtypeStruct((B * N,), jnp.float32),
        scratch_types=(
            [pltpu.VMEM((3, CHUNK), jnp.float32) for _ in range(NBUF)]
            + [pltpu.VMEM((3, CHUNK), jnp.float32) for _ in range(NBUF)]
            + [pltpu.VMEM((8 * CHUNK,), jnp.int32) for _ in range(NBUF)]
            + [pltpu.VMEM((8 * CHUNK,), jnp.float32) for _ in range(NBUF)]
            + [pltpu.VMEM((CHUNK,), jnp.float32) for _ in range(NBUF)]
            + [pltpu.SemaphoreType.DMA for _ in range(3 * NBUF)]
        ),
    )
    def sampler(grid_hbm, pts_hbm, out_hbm, *bufs):
        xyz = bufs[0:2]
        tbuf = bufs[2:4]
        idx = bufs[4:6]
        val = bufs[6:8]
        outv = bufs[8:10]
        sem_xyz = bufs[10:12]
        sem_g = bufs[12:14]
        sem_o = bufs[14:16]

        wid = lax.axis_index("s") * 2 + lax.axis_index("c")
        pbase = wid * N
        gbase = wid * NG3

        def start_xyz(ci, b):
            pltpu.async_copy(pts_hbm.at[:, pl.ds(pbase + ci * CHUNK, CHUNK)],
                             xyz[b], sem_xyz[b])

        def pass1(b):
            def grp(i, _):
                o = i * L
                xs = xyz[b][0, pl.ds(o, L)] + float(SCALE)
                ys = xyz[b][1, pl.ds(o, L)] + float(SCALE)
                zs = xyz[b][2, pl.ds(o, L)] + float(SCALE)
                fi = xs.astype(jnp.int32)
                fj = ys.astype(jnp.int32)
                fk = zs.astype(jnp.int32)
                tbuf[b][0, pl.ds(o, L)] = xs - fi.astype(jnp.float32)
                tbuf[b][1, pl.ds(o, L)] = ys - fj.astype(jnp.float32)
                tbuf[b][2, pl.ds(o, L)] = zs - fk.astype(jnp.float32)
                ii = jnp.minimum(fi, NG - 2)
                jj = jnp.minimum(fj, NG - 2)
                kk = jnp.minimum(fk, NG - 2)
                lin0 = (ii << 12) + (jj << 6) + kk + gbase
                for t in range(8):
                    idx[b][pl.ds(t * CHUNK + o, L)] = lin0 + _TAP_OFFS[t]

            lax.fori_loop(0, CHUNK // L, grp, None, unroll=2)

        def pass2(b):
            def grp(i, _):
                o = i * L
                tx = tbuf[b][0, pl.ds(o, L)]
                ty = tbuf[b][1, pl.ds(o, L)]
                tz = tbuf[b][2, pl.ds(o, L)]
                v = [val[b][pl.ds(t * CHUNK + o, L)] for t in range(8)]
                c00 = v[0] + tz * (v[1] - v[0])
                c01 = v[2] + tz * (v[3] - v[2])
                c10 = v[4] + tz * (v[5] - v[4])
                c11 = v[6] + tz * (v[7] - v[6])
                d0 = c00 + ty * (c01 - c00)
                d1 = c10 + ty * (c11 - c10)
                outv[b][pl.ds(o, L)] = d0 + tx * (d1 - d0)

            lax.fori_loop(0, CHUNK // L, grp, None, unroll=2)

        def start_gather(b):
            pltpu.async_copy(grid_hbm.at[idx[b]], val[b], sem_g[b])

        def start_out(ci, b):
            pltpu.async_copy(outv[b], out_hbm.at[pl.ds(pbase + ci * CHUNK, CHUNK)],
                             sem_o[b])

        def wait_xyz(b):
            pltpu.make_async_copy(pts_hbm.at[:, pl.ds(0, CHUNK)], xyz[b],
                                  sem_xyz[b]).wait()

        def wait_gather(b):
            pltpu.make_async_copy(grid_hbm.at[idx[b]], val[b], sem_g[b]).wait()

        def wait_out(b):
            pltpu.make_async_copy(outv[b], out_hbm.at[pl.ds(0, CHUNK)],
                                  sem_o[b]).wait()

        start_xyz(0, 0)
        start_xyz(1, 1)
        wait_xyz(0)
        pass1(0)
        start_gather(0)

        def step(ci, cur):
            nxt = 1 - cur
            @pl.when(ci + 1 < nch)
            def _():
                wait_xyz(nxt)
                pass1(nxt)
                start_gather(nxt)

            @pl.when(ci + 2 < nch)
            def _():
                start_xyz(ci + 2, cur)

            @pl.when(ci >= 2)
            def _():
                wait_out(cur)

            wait_gather(cur)
            pass2(cur)
            start_out(ci, cur)

        def two_steps(m, _):
            step(2 * m, 0)
            step(2 * m + 1, 1)
            return None

        lax.fori_loop(0, nch // 2, two_steps, None)
        wait_out(0)
        wait_out(1)

    return sampler


def kernel(grid, ptcloud):
    B, N = ptcloud.shape[0], ptcloud.shape[1]
    pts = ptcloud.reshape(B * N, 3).T
    out = _make_sampler(B, N)(grid.reshape(-1), pts)
    return out.reshape(B, N)
